# baseline (speedup 1.0000x reference)
import functools
import math

import jax
import jax.numpy as jnp
from jax.experimental import pallas as pl

_S = 64.0
_MARGIN = 0.5
_COS_M = math.cos(_MARGIN)
_SIN_M = math.sin(_MARGIN)

_RB = 256
_CB = 8192


def _arc_body(lab_ref, x_ref, out_ref, *, cb):
    j = pl.program_id(1)
    x = x_ref[...]
    lab = lab_ref[...]
    cols = j * cb + jax.lax.broadcasted_iota(jnp.int32, x.shape, 1)
    mask = cols == lab
    sin_theta = jnp.sqrt(jnp.maximum((1.0 - x) * (1.0 + x), 0.0))
    corrected = (_COS_M * x - _SIN_M * sin_theta) * _S
    out_ref[...] = jnp.where(mask, corrected, x * _S)


def kernel(logits, labels):
    rows, cols = logits.shape
    lab2 = labels.reshape(rows, 1)
    grid = (rows // _RB, pl.cdiv(cols, _CB))
    return pl.pallas_call(
        functools.partial(_arc_body, cb=_CB),
        grid=grid,
        in_specs=[
            pl.BlockSpec((_RB, 1), lambda i, j: (i, 0)),
            pl.BlockSpec((_RB, _CB), lambda i, j: (i, j)),
        ],
        out_specs=pl.BlockSpec((_RB, _CB), lambda i, j: (i, j)),
        out_shape=jax.ShapeDtypeStruct((rows, cols), jnp.float32),
    )(lab2, logits)


# speedup vs baseline: 1.0212x; 1.0212x over previous
import functools
import math

import jax
import jax.numpy as jnp
from jax.experimental import pallas as pl

_S = 64.0
_MARGIN = 0.5
_COS_M = math.cos(_MARGIN)
_SIN_M = math.sin(_MARGIN)

_RB = 1024
_CB = 2048


def _arc_body(lab_ref, x_ref, out_ref, *, cb):
    j = pl.program_id(1)
    x = x_ref[...]
    lab = lab_ref[...]
    cols = j * cb + jax.lax.broadcasted_iota(jnp.int32, x.shape, 1)
    mask = cols == lab
    sin_theta = jnp.sqrt(jnp.maximum((1.0 - x) * (1.0 + x), 0.0))
    corrected = (_COS_M * x - _SIN_M * sin_theta) * _S
    out_ref[...] = jnp.where(mask, corrected, x * _S)


def kernel(logits, labels):
    rows, cols = logits.shape
    lab2 = labels.reshape(rows, 1)
    grid = (rows // _RB, pl.cdiv(cols, _CB))
    return pl.pallas_call(
        functools.partial(_arc_body, cb=_CB),
        grid=grid,
        in_specs=[
            pl.BlockSpec((_RB, 1), lambda i, j: (i, 0)),
            pl.BlockSpec((_RB, _CB), lambda i, j: (i, j)),
        ],
        out_specs=pl.BlockSpec((_RB, _CB), lambda i, j: (i, j)),
        out_shape=jax.ShapeDtypeStruct((rows, cols), jnp.float32),
    )(lab2, logits)
